# baseline (speedup 1.0000x reference)
import functools

import numpy as np
import jax
import jax.numpy as jnp
from jax import lax
from jax.experimental import pallas as pl
from jax.experimental.pallas import tpu as pltpu
from jax.experimental.pallas import tpu_sc as plsc

N = 10000
F_IN = 128
NEG = 0.2
NC = 40

NP_ = 10112
RPT = NP_ // 16
NWORK = 32
CHUNK = 128
CPW = 80
EW = CHUNK * CPW
E_PAD = EW * NWORK

_F32 = jnp.float32


def _perm64():
    r = lax.broadcasted_iota(jnp.int32, (64, 64), 0)
    c = lax.broadcasted_iota(jnp.int32, (64, 64), 1)
    return ((c % 8) * 8 + c // 8 == r).astype(_F32)


def _head_proj(a):
    h = lax.broadcasted_iota(jnp.int32, (8, 8, 8), 0)
    k = lax.broadcasted_iota(jnp.int32, (8, 8, 8), 2)
    eye = (h == k).astype(_F32)
    return (a[:, :, None] * eye).reshape(64, 8)


def _elu(x):
    return jnp.where(x > 0, x, jnp.exp(x) - 1.0)


def _den_expand(q):
    outt = q[:, :64]
    den = q[:, 64:72]
    denx = jnp.tile(den, (1, 8))
    return jnp.where(denx > 0, outt / denx, 0.0)


BR = NP_ // 8


def _tc0_body(x_ref, w0_ref, al_ref, ar_ref, hx_ref, er_ref):
    P = _perm64()
    h = jnp.dot(x_ref[...], w0_ref[...], preferred_element_type=_F32)
    el = jnp.dot(h, _head_proj(al_ref[...]), preferred_element_type=_F32)
    er = jnp.dot(h, _head_proj(ar_ref[...]), preferred_element_type=_F32)
    ht = jnp.dot(h, P, preferred_element_type=_F32)
    hx_ref[...] = jnp.concatenate([ht, el, el], axis=1)
    er_ref[...] = jnp.concatenate([er, er], axis=1)


def _tc1_body(p_ref, b0_ref, w1_ref, al_ref, ar_ref, hx_ref, er_ref, h0t_ref):
    P = _perm64()
    q = p_ref[0] + p_ref[1]
    rstt = _den_expand(q)
    b0t = jnp.dot(b0_ref[...][None, :], P, preferred_element_type=_F32)
    h0t = _elu(rstt + b0t)
    h0t_ref[...] = h0t
    w1p = jnp.dot(P, w1_ref[...], preferred_element_type=_F32)
    h1 = jnp.dot(h0t, w1p, preferred_element_type=_F32)
    el = jnp.dot(h1, _head_proj(al_ref[...]), preferred_element_type=_F32)
    er = jnp.dot(h1, _head_proj(ar_ref[...]), preferred_element_type=_F32)
    h1t = jnp.dot(h1, P, preferred_element_type=_F32)
    hx_ref[...] = jnp.concatenate([h1t, el, el], axis=1)
    er_ref[...] = jnp.concatenate([er, er], axis=1)


def _tc2_body(p_ref, h0t_ref, b1_ref, w2_ref, wres2_ref, al2_ref, ar2_ref,
              hx2_ref, er2_ref, res2_ref):
    P = _perm64()
    q = p_ref[0] + p_ref[1]
    rstt = _den_expand(q)
    b1t = jnp.dot(b1_ref[...][None, :], P, preferred_element_type=_F32)
    h1t = _elu(rstt + h0t_ref[...] + b1t)
    w2p = jnp.dot(P, w2_ref[...], preferred_element_type=_F32)
    h2 = jnp.dot(h1t, w2p, preferred_element_type=_F32)
    el2 = jnp.dot(h2, al2_ref[...].reshape(8, 1), preferred_element_type=_F32)
    er2 = jnp.dot(h2, ar2_ref[...].reshape(8, 1), preferred_element_type=_F32)
    wres2p = jnp.dot(P, wres2_ref[...], preferred_element_type=_F32)
    res2_ref[...] = jnp.dot(h1t, wres2p, preferred_element_type=_F32)
    hx2_ref[...] = jnp.concatenate([h2, jnp.tile(el2, (1, 8))], axis=1)
    er2_ref[...] = jnp.tile(er2, (1, 16))


def _tc3_body(p_ref, res2_ref, b2_ref, wfc_ref, bfc_ref, out_ref):
    q = p_ref[0] + p_ref[1]
    outv = q[:, :8]
    den = q[:, 8:9]
    h2 = jnp.where(den > 0, outv / den, 0.0) + res2_ref[...] + b2_ref[...][None, :]
    out_ref[...] = (jnp.dot(h2, wfc_ref[...], preferred_element_type=_F32)
                    + bfc_ref[...][None, :])


def _row_spec(shape):
    if shape[0] == NP_:
        blk = (BR,) + shape[1:]
        return pl.BlockSpec(blk, lambda i: (i,) + (0,) * (len(shape) - 1))
    if len(shape) == 3 and shape[1] == NP_:
        blk = (shape[0], BR, shape[2])
        return pl.BlockSpec(blk, lambda i: (0, i, 0))
    return pl.BlockSpec(shape, lambda i: (0,) * len(shape))


def _tc_call(body, out_shapes, *args):
    if not isinstance(out_shapes, tuple):
        out_specs = _row_spec(out_shapes.shape)
    else:
        out_specs = tuple(_row_spec(o.shape) for o in out_shapes)
    return pl.pallas_call(
        body, out_shape=out_shapes,
        grid=(NP_ // BR,),
        in_specs=[_row_spec(a.shape) for a in args],
        out_specs=out_specs,
        compiler_params=pltpu.CompilerParams(
            dimension_semantics=("arbitrary",)),
    )(*args)


_MESH = plsc.VectorSubcoreMesh(core_axis_name="c", subcore_axis_name="s")


def _lane_gather(w, idx):
    dn = lax.GatherDimensionNumbers(
        offset_dims=(), collapsed_slice_dims=(0,), start_index_map=(0,))
    return lax.gather(w, idx[:, None], dn, slice_sizes=(1,),
                      mode=lax.GatherScatterMode.PROMISE_IN_BOUNDS)


def _edge_compute80(rows, erv):
    def edge_body(e, carry):
        vel = rows[e, pl.ds(64, 16)]
        ver = erv[e, :]
        sv = vel + ver
        w = jnp.exp(jnp.maximum(sv, sv * NEG))
        rows[e, pl.ds(64, 16)] = w
        for i in range(4):
            rows[e, pl.ds(16 * i, 16)] = rows[e, pl.ds(16 * i, 16)] * w
        return carry

    lax.fori_loop(0, CHUNK, edge_body, 0, unroll=4)


def _edge_compute16(rows, erv):
    lane = lax.iota(jnp.int32, 16)
    lt8 = lane < 8
    idx8 = lane * 0 + 8

    def edge_body(e, carry):
        vh = rows[e, :]
        ver = erv[e, :]
        sv = vh + ver
        w = jnp.exp(jnp.maximum(sv, sv * NEG))
        wf = _lane_gather(w, idx8)
        rows[e, :] = jnp.where(lt8, vh, 1.0) * wf
        return carry

    lax.fori_loop(0, CHUNK, edge_body, 0, unroll=4)


NBUF = 4


def _make_sc_edge(row_w, compute):
    scratch = [
        pltpu.VMEM((CPW + NBUF, CHUNK), jnp.int32),
        pltpu.VMEM((CPW + NBUF, CHUNK), jnp.int32),
    ]
    scratch += [pltpu.VMEM((CHUNK, row_w), _F32) for _ in range(NBUF)]
    scratch += [pltpu.VMEM((CHUNK, 16), _F32) for _ in range(NBUF)]
    scratch += [pltpu.VMEM_SHARED((NP_, row_w), _F32)]
    scratch += [pltpu.SemaphoreType.DMA] * (3 * NBUF)

    @functools.partial(
        pl.kernel,
        out_type=jax.ShapeDtypeStruct((2, NP_, row_w), _F32),
        mesh=_MESH,
        scratch_types=scratch,
        compiler_params=pltpu.CompilerParams(use_tc_tiling_on_sc=False),
    )
    def sc_edge(hx_hbm, er_hbm, src_hbm, dst_hbm, z_hbm, out_hbm,
                srcall, dstall, *bufs):
        rows = bufs[0:NBUF]
        erv = bufs[NBUF:2 * NBUF]
        acc = bufs[2 * NBUF]
        semH = bufs[2 * NBUF + 1:2 * NBUF + 1 + NBUF]
        semE = bufs[2 * NBUF + 1 + NBUF:2 * NBUF + 1 + 2 * NBUF]
        semS = bufs[2 * NBUF + 1 + 2 * NBUF:2 * NBUF + 1 + 3 * NBUF]
        c = lax.axis_index("c")
        s = lax.axis_index("s")
        wid = c * 16 + s
        pltpu.sync_copy(z_hbm.at[pl.ds(s * RPT, RPT)], acc.at[pl.ds(s * RPT, RPT)])

        brow = wid * CPW
        pltpu.sync_copy(src_hbm.at[pl.ds(brow, CPW)], srcall.at[pl.ds(0, CPW)])
        pltpu.sync_copy(dst_hbm.at[pl.ds(brow, CPW)], dstall.at[pl.ds(0, CPW)])
        for k in range(NBUF):
            pltpu.sync_copy(src_hbm.at[brow], srcall.at[CPW + k])
            pltpu.sync_copy(dst_hbm.at[brow], dstall.at[CPW + k])
        plsc.subcore_barrier()

        for k in range(NBUF):
            pltpu.async_copy(hx_hbm.at[srcall.at[k]], rows[k], semH[k])
            pltpu.async_copy(er_hbm.at[dstall.at[k]], erv[k], semE[k])

        def group_body(j, carry):
            cbase = NBUF * j
            for k in range(NBUF):
                pltpu.make_async_copy(
                    hx_hbm.at[srcall.at[cbase + k]], rows[k], semH[k]).wait()
                pltpu.make_async_copy(
                    er_hbm.at[dstall.at[cbase + k]], erv[k], semE[k]).wait()
                compute(rows[k], erv[k])
                pltpu.async_copy(rows[k], acc.at[dstall.at[cbase + k]],
                                 semS[k], add=True)
            for k in range(NBUF):
                pltpu.make_async_copy(
                    rows[k], acc.at[dstall.at[cbase + k]], semS[k]).wait()
                nxt = cbase + NBUF + k
                pltpu.async_copy(hx_hbm.at[srcall.at[nxt]], rows[k], semH[k])
                pltpu.async_copy(er_hbm.at[dstall.at[nxt]], erv[k], semE[k])
            return carry

        lax.fori_loop(0, CPW // NBUF, group_body, 0)
        for k in range(NBUF):
            pltpu.make_async_copy(
                hx_hbm.at[srcall.at[CPW + k]], rows[k], semH[k]).wait()
            pltpu.make_async_copy(
                er_hbm.at[dstall.at[CPW + k]], erv[k], semE[k]).wait()

        plsc.subcore_barrier()
        pltpu.sync_copy(acc.at[pl.ds(s * RPT, RPT)],
                        out_hbm.at[c, pl.ds(s * RPT, RPT)])

    return sc_edge


_sc_edge80 = _make_sc_edge(80, _edge_compute80)
_sc_edge16 = _make_sc_edge(16, _edge_compute16)


def kernel(x, edge_index, W0, al0, ar0, b0, W1, al1, ar1, b1,
           W2, al2, ar2, b2, Wres2, Wfc, bfc):
    E = edge_index.shape[1]
    pad_e = E_PAD - E
    pad_idx = (N + jnp.arange(pad_e, dtype=jnp.int32) % (NP_ - N)).astype(jnp.int32)
    src = jnp.concatenate([edge_index[0], pad_idx]).reshape(E_PAD // CHUNK, CHUNK)
    dst = jnp.concatenate([edge_index[1], pad_idx]).reshape(E_PAD // CHUNK, CHUNK)
    x_pad = jnp.pad(x, ((0, NP_ - N), (0, 0)))
    z80 = jnp.zeros((NP_, 80), _F32)
    z16 = jnp.zeros((NP_, 16), _F32)

    hx0, er0 = _tc_call(
        _tc0_body,
        (jax.ShapeDtypeStruct((NP_, 80), _F32), jax.ShapeDtypeStruct((NP_, 16), _F32)),
        x_pad, W0, al0, ar0)
    p0 = _sc_edge80(hx0, er0, src, dst, z80)

    hx1, er1, h0t = _tc_call(
        _tc1_body,
        (jax.ShapeDtypeStruct((NP_, 80), _F32), jax.ShapeDtypeStruct((NP_, 16), _F32),
         jax.ShapeDtypeStruct((NP_, 64), _F32)),
        p0, b0, W1, al1, ar1)
    p1 = _sc_edge80(hx1, er1, src, dst, z80)

    hx2, er2, res2 = _tc_call(
        _tc2_body,
        (jax.ShapeDtypeStruct((NP_, 16), _F32), jax.ShapeDtypeStruct((NP_, 16), _F32),
         jax.ShapeDtypeStruct((NP_, 8), _F32)),
        p1, h0t, b1, W2, Wres2, al2, ar2)
    p2 = _sc_edge16(hx2, er2, src, dst, z16)

    logits = _tc_call(
        _tc3_body,
        jax.ShapeDtypeStruct((NP_, NC), _F32),
        p2, res2, b2, Wfc, bfc)
    return logits[:N]


# speedup vs baseline: 1.8493x; 1.8493x over previous
import functools

import numpy as np
import jax
import jax.numpy as jnp
from jax import lax
from jax.experimental import pallas as pl
from jax.experimental.pallas import tpu as pltpu
from jax.experimental.pallas import tpu_sc as plsc

N = 10000
F_IN = 128
NEG = 0.2
NC = 40

NP_ = 10112
RPT = NP_ // 16
NWORK = 32
CHUNK = 128
CPW = 80
EW = CHUNK * CPW
E_PAD = EW * NWORK

_F32 = jnp.float32


def _perm64():
    r = lax.broadcasted_iota(jnp.int32, (64, 64), 0)
    c = lax.broadcasted_iota(jnp.int32, (64, 64), 1)
    return ((c % 8) * 8 + c // 8 == r).astype(_F32)


def _head_proj(a):
    h = lax.broadcasted_iota(jnp.int32, (8, 8, 8), 0)
    k = lax.broadcasted_iota(jnp.int32, (8, 8, 8), 2)
    eye = (h == k).astype(_F32)
    return (a[:, :, None] * eye).reshape(64, 8)


def _elu(x):
    return jnp.where(x > 0, x, jnp.exp(x) - 1.0)


def _den_expand(q):
    outt = q[:, :64]
    den = q[:, 64:72]
    denx = jnp.tile(den, (1, 8))
    return jnp.where(denx > 0, outt / denx, 0.0)


BR = NP_ // 8


def _tc0_body(x_ref, w0_ref, al_ref, ar_ref, hx_ref, er_ref):
    P = _perm64()
    h = jnp.dot(x_ref[...], w0_ref[...], preferred_element_type=_F32)
    el = jnp.dot(h, _head_proj(al_ref[...]), preferred_element_type=_F32)
    er = jnp.dot(h, _head_proj(ar_ref[...]), preferred_element_type=_F32)
    ht = jnp.dot(h, P, preferred_element_type=_F32)
    hx_ref[...] = jnp.concatenate([ht, el, el], axis=1)
    er_ref[...] = jnp.concatenate([er, er], axis=1)


def _tc1_body(p_ref, b0_ref, w1_ref, al_ref, ar_ref, hx_ref, er_ref, h0t_ref):
    P = _perm64()
    q = p_ref[0] + p_ref[1]
    rstt = _den_expand(q)
    b0t = jnp.dot(b0_ref[...][None, :], P, preferred_element_type=_F32)
    h0t = _elu(rstt + b0t)
    h0t_ref[...] = h0t
    w1p = jnp.dot(P, w1_ref[...], preferred_element_type=_F32)
    h1 = jnp.dot(h0t, w1p, preferred_element_type=_F32)
    el = jnp.dot(h1, _head_proj(al_ref[...]), preferred_element_type=_F32)
    er = jnp.dot(h1, _head_proj(ar_ref[...]), preferred_element_type=_F32)
    h1t = jnp.dot(h1, P, preferred_element_type=_F32)
    hx_ref[...] = jnp.concatenate([h1t, el, el], axis=1)
    er_ref[...] = jnp.concatenate([er, er], axis=1)


def _tc2_body(p_ref, h0t_ref, b1_ref, w2_ref, wres2_ref, al2_ref, ar2_ref,
              hx2_ref, er2_ref, res2_ref):
    P = _perm64()
    q = p_ref[0] + p_ref[1]
    rstt = _den_expand(q)
    b1t = jnp.dot(b1_ref[...][None, :], P, preferred_element_type=_F32)
    h1t = _elu(rstt + h0t_ref[...] + b1t)
    w2p = jnp.dot(P, w2_ref[...], preferred_element_type=_F32)
    h2 = jnp.dot(h1t, w2p, preferred_element_type=_F32)
    el2 = jnp.dot(h2, al2_ref[...].reshape(8, 1), preferred_element_type=_F32)
    er2 = jnp.dot(h2, ar2_ref[...].reshape(8, 1), preferred_element_type=_F32)
    wres2p = jnp.dot(P, wres2_ref[...], preferred_element_type=_F32)
    res2_ref[...] = jnp.dot(h1t, wres2p, preferred_element_type=_F32)
    hx2_ref[...] = jnp.concatenate([h2, jnp.tile(el2, (1, 8))], axis=1)
    er2_ref[...] = jnp.tile(er2, (1, 16))


def _tc3_body(p_ref, res2_ref, b2_ref, wfc_ref, bfc_ref, out_ref):
    q = p_ref[0] + p_ref[1]
    outv = q[:, :8]
    den = q[:, 8:9]
    h2 = jnp.where(den > 0, outv / den, 0.0) + res2_ref[...] + b2_ref[...][None, :]
    out_ref[...] = (jnp.dot(h2, wfc_ref[...], preferred_element_type=_F32)
                    + bfc_ref[...][None, :])


def _row_spec(shape):
    if shape[0] == NP_:
        blk = (BR,) + shape[1:]
        return pl.BlockSpec(blk, lambda i: (i,) + (0,) * (len(shape) - 1))
    if len(shape) == 3 and shape[1] == NP_:
        blk = (shape[0], BR, shape[2])
        return pl.BlockSpec(blk, lambda i: (0, i, 0))
    return pl.BlockSpec(shape, lambda i: (0,) * len(shape))


def _tc_call(body, out_shapes, *args):
    if not isinstance(out_shapes, tuple):
        out_specs = _row_spec(out_shapes.shape)
    else:
        out_specs = tuple(_row_spec(o.shape) for o in out_shapes)
    return pl.pallas_call(
        body, out_shape=out_shapes,
        grid=(NP_ // BR,),
        in_specs=[_row_spec(a.shape) for a in args],
        out_specs=out_specs,
        compiler_params=pltpu.CompilerParams(
            dimension_semantics=("arbitrary",)),
    )(*args)


_MESH = plsc.VectorSubcoreMesh(core_axis_name="c", subcore_axis_name="s")


def _lane_gather(w, idx):
    dn = lax.GatherDimensionNumbers(
        offset_dims=(), collapsed_slice_dims=(0,), start_index_map=(0,))
    return lax.gather(w, idx[:, None], dn, slice_sizes=(1,),
                      mode=lax.GatherScatterMode.PROMISE_IN_BOUNDS)


def _edge_compute80(rows, erv):
    @plsc.parallel_loop(0, CHUNK, unroll=8)
    def edge_body(e):
        vel = rows[e, pl.ds(64, 16)]
        ver = erv[e, :]
        sv = vel + ver
        w = jnp.exp(jnp.maximum(sv, sv * NEG))
        rows[e, pl.ds(64, 16)] = w
        for i in range(4):
            rows[e, pl.ds(16 * i, 16)] = rows[e, pl.ds(16 * i, 16)] * w


def _edge_compute16(rows, erv):
    lane = lax.iota(jnp.int32, 16)
    lt8 = lane < 8
    idx8 = lane * 0 + 8

    @plsc.parallel_loop(0, CHUNK, unroll=8)
    def edge_body(e):
        vh = rows[e, :]
        ver = erv[e, :]
        sv = vh + ver
        w = jnp.exp(jnp.maximum(sv, sv * NEG))
        wf = _lane_gather(w, idx8)
        rows[e, :] = jnp.where(lt8, vh, 1.0) * wf


NBUF = 4


def _make_sc_edge(row_w, compute):
    scratch = [
        pltpu.VMEM((CPW + NBUF, CHUNK), jnp.int32),
        pltpu.VMEM((CPW + NBUF, CHUNK), jnp.int32),
    ]
    scratch += [pltpu.VMEM((CHUNK, row_w), _F32) for _ in range(NBUF)]
    scratch += [pltpu.VMEM((CHUNK, 16), _F32) for _ in range(NBUF)]
    scratch += [pltpu.VMEM_SHARED((NP_, row_w), _F32)]
    scratch += [pltpu.SemaphoreType.DMA] * (3 * NBUF)

    @functools.partial(
        pl.kernel,
        out_type=jax.ShapeDtypeStruct((2, NP_, row_w), _F32),
        mesh=_MESH,
        scratch_types=scratch,
        compiler_params=pltpu.CompilerParams(use_tc_tiling_on_sc=False),
    )
    def sc_edge(hx_hbm, er_hbm, src_hbm, dst_hbm, z_hbm, out_hbm,
                srcall, dstall, *bufs):
        rows = bufs[0:NBUF]
        erv = bufs[NBUF:2 * NBUF]
        acc = bufs[2 * NBUF]
        semH = bufs[2 * NBUF + 1:2 * NBUF + 1 + NBUF]
        semE = bufs[2 * NBUF + 1 + NBUF:2 * NBUF + 1 + 2 * NBUF]
        semS = bufs[2 * NBUF + 1 + 2 * NBUF:2 * NBUF + 1 + 3 * NBUF]
        c = lax.axis_index("c")
        s = lax.axis_index("s")
        wid = c * 16 + s
        pltpu.sync_copy(z_hbm.at[pl.ds(s * RPT, RPT)], acc.at[pl.ds(s * RPT, RPT)])

        brow = wid * CPW
        pltpu.sync_copy(src_hbm.at[pl.ds(brow, CPW)], srcall.at[pl.ds(0, CPW)])
        pltpu.sync_copy(dst_hbm.at[pl.ds(brow, CPW)], dstall.at[pl.ds(0, CPW)])
        for k in range(NBUF):
            pltpu.sync_copy(src_hbm.at[brow], srcall.at[CPW + k])
            pltpu.sync_copy(dst_hbm.at[brow], dstall.at[CPW + k])
        plsc.subcore_barrier()

        for k in range(NBUF):
            pltpu.async_copy(hx_hbm.at[srcall.at[k]], rows[k], semH[k])
            pltpu.async_copy(er_hbm.at[dstall.at[k]], erv[k], semE[k])

        def group_body(j, carry):
            cbase = NBUF * j
            for k in range(NBUF):
                pltpu.make_async_copy(
                    hx_hbm.at[srcall.at[cbase + k]], rows[k], semH[k]).wait()
                pltpu.make_async_copy(
                    er_hbm.at[dstall.at[cbase + k]], erv[k], semE[k]).wait()
                compute(rows[k], erv[k])
                pltpu.async_copy(rows[k], acc.at[dstall.at[cbase + k]],
                                 semS[k], add=True)
            for k in range(NBUF):
                pltpu.make_async_copy(
                    rows[k], acc.at[dstall.at[cbase + k]], semS[k]).wait()
                nxt = cbase + NBUF + k
                pltpu.async_copy(hx_hbm.at[srcall.at[nxt]], rows[k], semH[k])
                pltpu.async_copy(er_hbm.at[dstall.at[nxt]], erv[k], semE[k])
            return carry

        lax.fori_loop(0, CPW // NBUF, group_body, 0)
        for k in range(NBUF):
            pltpu.make_async_copy(
                hx_hbm.at[srcall.at[CPW + k]], rows[k], semH[k]).wait()
            pltpu.make_async_copy(
                er_hbm.at[dstall.at[CPW + k]], erv[k], semE[k]).wait()

        plsc.subcore_barrier()
        pltpu.sync_copy(acc.at[pl.ds(s * RPT, RPT)],
                        out_hbm.at[c, pl.ds(s * RPT, RPT)])

    return sc_edge


_sc_edge80 = _make_sc_edge(80, _edge_compute80)
_sc_edge16 = _make_sc_edge(16, _edge_compute16)


def kernel(x, edge_index, W0, al0, ar0, b0, W1, al1, ar1, b1,
           W2, al2, ar2, b2, Wres2, Wfc, bfc):
    E = edge_index.shape[1]
    pad_e = E_PAD - E
    pad_idx = (N + jnp.arange(pad_e, dtype=jnp.int32) % (NP_ - N)).astype(jnp.int32)
    src = jnp.concatenate([edge_index[0], pad_idx]).reshape(E_PAD // CHUNK, CHUNK)
    dst = jnp.concatenate([edge_index[1], pad_idx]).reshape(E_PAD // CHUNK, CHUNK)
    x_pad = jnp.pad(x, ((0, NP_ - N), (0, 0)))
    z80 = jnp.zeros((NP_, 80), _F32)
    z16 = jnp.zeros((NP_, 16), _F32)

    hx0, er0 = _tc_call(
        _tc0_body,
        (jax.ShapeDtypeStruct((NP_, 80), _F32), jax.ShapeDtypeStruct((NP_, 16), _F32)),
        x_pad, W0, al0, ar0)
    p0 = _sc_edge80(hx0, er0, src, dst, z80)

    hx1, er1, h0t = _tc_call(
        _tc1_body,
        (jax.ShapeDtypeStruct((NP_, 80), _F32), jax.ShapeDtypeStruct((NP_, 16), _F32),
         jax.ShapeDtypeStruct((NP_, 64), _F32)),
        p0, b0, W1, al1, ar1)
    p1 = _sc_edge80(hx1, er1, src, dst, z80)

    hx2, er2, res2 = _tc_call(
        _tc2_body,
        (jax.ShapeDtypeStruct((NP_, 16), _F32), jax.ShapeDtypeStruct((NP_, 16), _F32),
         jax.ShapeDtypeStruct((NP_, 8), _F32)),
        p1, h0t, b1, W2, Wres2, al2, ar2)
    p2 = _sc_edge16(hx2, er2, src, dst, z16)

    logits = _tc_call(
        _tc3_body,
        jax.ShapeDtypeStruct((NP_, NC), _F32),
        p2, res2, b2, Wfc, bfc)
    return logits[:N]


# speedup vs baseline: 1.8645x; 1.0082x over previous
import functools

import numpy as np
import jax
import jax.numpy as jnp
from jax import lax
from jax.experimental import pallas as pl
from jax.experimental.pallas import tpu as pltpu
from jax.experimental.pallas import tpu_sc as plsc

N = 10000
F_IN = 128
NEG = 0.2
NC = 40

NP_ = 10112
RPT = NP_ // 16
NWORK = 32
CHUNK = 128
CPW = 80
EW = CHUNK * CPW
E_PAD = EW * NWORK

_F32 = jnp.float32


def _perm64():
    r = lax.broadcasted_iota(jnp.int32, (64, 64), 0)
    c = lax.broadcasted_iota(jnp.int32, (64, 64), 1)
    return ((c % 8) * 8 + c // 8 == r).astype(_F32)


def _head_proj(a):
    h = lax.broadcasted_iota(jnp.int32, (8, 8, 8), 0)
    k = lax.broadcasted_iota(jnp.int32, (8, 8, 8), 2)
    eye = (h == k).astype(_F32)
    return (a[:, :, None] * eye).reshape(64, 8)


def _elu(x):
    return jnp.where(x > 0, x, jnp.exp(x) - 1.0)


def _den_expand(q):
    outt = q[:, :64]
    den = q[:, 64:72]
    denx = jnp.tile(den, (1, 8))
    return jnp.where(denx > 0, outt / denx, 0.0)


BR = NP_ // 4


def _tc0_body(x_ref, w0_ref, al_ref, ar_ref, hx_ref, er_ref):
    P = _perm64()
    h = jnp.dot(x_ref[...], w0_ref[...], preferred_element_type=_F32)
    el = jnp.dot(h, _head_proj(al_ref[...]), preferred_element_type=_F32)
    er = jnp.dot(h, _head_proj(ar_ref[...]), preferred_element_type=_F32)
    ht = jnp.dot(h, P, preferred_element_type=_F32)
    hx_ref[...] = jnp.concatenate([ht, el, el], axis=1)
    er_ref[...] = jnp.concatenate([er, er], axis=1)


def _tc1_body(p_ref, b0_ref, w1_ref, al_ref, ar_ref, hx_ref, er_ref, h0t_ref):
    P = _perm64()
    q = p_ref[0] + p_ref[1]
    rstt = _den_expand(q)
    b0t = jnp.dot(b0_ref[...][None, :], P, preferred_element_type=_F32)
    h0t = _elu(rstt + b0t)
    h0t_ref[...] = h0t
    w1p = jnp.dot(P, w1_ref[...], preferred_element_type=_F32)
    h1 = jnp.dot(h0t, w1p, preferred_element_type=_F32)
    el = jnp.dot(h1, _head_proj(al_ref[...]), preferred_element_type=_F32)
    er = jnp.dot(h1, _head_proj(ar_ref[...]), preferred_element_type=_F32)
    h1t = jnp.dot(h1, P, preferred_element_type=_F32)
    hx_ref[...] = jnp.concatenate([h1t, el, el], axis=1)
    er_ref[...] = jnp.concatenate([er, er], axis=1)


def _tc2_body(p_ref, h0t_ref, b1_ref, w2_ref, wres2_ref, al2_ref, ar2_ref,
              hx2_ref, er2_ref, res2_ref):
    P = _perm64()
    q = p_ref[0] + p_ref[1]
    rstt = _den_expand(q)
    b1t = jnp.dot(b1_ref[...][None, :], P, preferred_element_type=_F32)
    h1t = _elu(rstt + h0t_ref[...] + b1t)
    w2p = jnp.dot(P, w2_ref[...], preferred_element_type=_F32)
    h2 = jnp.dot(h1t, w2p, preferred_element_type=_F32)
    el2 = jnp.dot(h2, al2_ref[...].reshape(8, 1), preferred_element_type=_F32)
    er2 = jnp.dot(h2, ar2_ref[...].reshape(8, 1), preferred_element_type=_F32)
    wres2p = jnp.dot(P, wres2_ref[...], preferred_element_type=_F32)
    res2_ref[...] = jnp.dot(h1t, wres2p, preferred_element_type=_F32)
    hx2_ref[...] = jnp.concatenate([h2, jnp.tile(el2, (1, 8))], axis=1)
    er2_ref[...] = jnp.tile(er2, (1, 16))


def _tc3_body(p_ref, res2_ref, b2_ref, wfc_ref, bfc_ref, out_ref):
    q = p_ref[0] + p_ref[1]
    outv = q[:, :8]
    den = q[:, 8:9]
    h2 = jnp.where(den > 0, outv / den, 0.0) + res2_ref[...] + b2_ref[...][None, :]
    out_ref[...] = (jnp.dot(h2, wfc_ref[...], preferred_element_type=_F32)
                    + bfc_ref[...][None, :])


def _row_spec(shape):
    if shape[0] == NP_:
        blk = (BR,) + shape[1:]
        return pl.BlockSpec(blk, lambda i: (i,) + (0,) * (len(shape) - 1))
    if len(shape) == 3 and shape[1] == NP_:
        blk = (shape[0], BR, shape[2])
        return pl.BlockSpec(blk, lambda i: (0, i, 0))
    return pl.BlockSpec(shape, lambda i: (0,) * len(shape))


def _tc_call(body, out_shapes, *args):
    if not isinstance(out_shapes, tuple):
        out_specs = _row_spec(out_shapes.shape)
    else:
        out_specs = tuple(_row_spec(o.shape) for o in out_shapes)
    return pl.pallas_call(
        body, out_shape=out_shapes,
        grid=(NP_ // BR,),
        in_specs=[_row_spec(a.shape) for a in args],
        out_specs=out_specs,
        compiler_params=pltpu.CompilerParams(
            dimension_semantics=("arbitrary",)),
    )(*args)


_MESH = plsc.VectorSubcoreMesh(core_axis_name="c", subcore_axis_name="s")


def _lane_gather(w, idx):
    dn = lax.GatherDimensionNumbers(
        offset_dims=(), collapsed_slice_dims=(0,), start_index_map=(0,))
    return lax.gather(w, idx[:, None], dn, slice_sizes=(1,),
                      mode=lax.GatherScatterMode.PROMISE_IN_BOUNDS)


def _edge_compute80(rows, erv):
    @plsc.parallel_loop(0, CHUNK, unroll=8)
    def edge_body(e):
        vel = rows[e, pl.ds(64, 16)]
        ver = erv[e, :]
        sv = vel + ver
        w = jnp.exp(jnp.maximum(sv, sv * NEG))
        rows[e, pl.ds(64, 16)] = w
        for i in range(4):
            rows[e, pl.ds(16 * i, 16)] = rows[e, pl.ds(16 * i, 16)] * w


def _edge_compute16(rows, erv):
    lane = lax.iota(jnp.int32, 16)
    lt8 = lane < 8
    idx8 = lane * 0 + 8

    @plsc.parallel_loop(0, CHUNK, unroll=8)
    def edge_body(e):
        vh = rows[e, :]
        ver = erv[e, :]
        sv = vh + ver
        w = jnp.exp(jnp.maximum(sv, sv * NEG))
        wf = _lane_gather(w, idx8)
        rows[e, :] = jnp.where(lt8, vh, 1.0) * wf


NBUF = 4


def _make_sc_edge(row_w, compute):
    scratch = [
        pltpu.VMEM((CPW + NBUF, CHUNK), jnp.int32),
        pltpu.VMEM((CPW + NBUF, CHUNK), jnp.int32),
    ]
    scratch += [pltpu.VMEM((CHUNK, row_w), _F32) for _ in range(NBUF)]
    scratch += [pltpu.VMEM((CHUNK, 16), _F32) for _ in range(NBUF)]
    scratch += [pltpu.VMEM_SHARED((NP_, row_w), _F32)]
    scratch += [pltpu.SemaphoreType.DMA] * (3 * NBUF)

    @functools.partial(
        pl.kernel,
        out_type=jax.ShapeDtypeStruct((2, NP_, row_w), _F32),
        mesh=_MESH,
        scratch_types=scratch,
        compiler_params=pltpu.CompilerParams(use_tc_tiling_on_sc=False),
    )
    def sc_edge(hx_hbm, er_hbm, src_hbm, dst_hbm, z_hbm, out_hbm,
                srcall, dstall, *bufs):
        rows = bufs[0:NBUF]
        erv = bufs[NBUF:2 * NBUF]
        acc = bufs[2 * NBUF]
        semH = bufs[2 * NBUF + 1:2 * NBUF + 1 + NBUF]
        semE = bufs[2 * NBUF + 1 + NBUF:2 * NBUF + 1 + 2 * NBUF]
        semS = bufs[2 * NBUF + 1 + 2 * NBUF:2 * NBUF + 1 + 3 * NBUF]
        c = lax.axis_index("c")
        s = lax.axis_index("s")
        wid = c * 16 + s
        pltpu.sync_copy(z_hbm.at[pl.ds(s * RPT, RPT)], acc.at[pl.ds(s * RPT, RPT)])

        brow = wid * CPW
        pltpu.sync_copy(src_hbm.at[pl.ds(brow, CPW)], srcall.at[pl.ds(0, CPW)])
        pltpu.sync_copy(dst_hbm.at[pl.ds(brow, CPW)], dstall.at[pl.ds(0, CPW)])
        for k in range(NBUF):
            pltpu.sync_copy(src_hbm.at[brow], srcall.at[CPW + k])
            pltpu.sync_copy(dst_hbm.at[brow], dstall.at[CPW + k])
        plsc.subcore_barrier()

        for k in range(NBUF):
            pltpu.async_copy(hx_hbm.at[srcall.at[k]], rows[k], semH[k])
            pltpu.async_copy(er_hbm.at[dstall.at[k]], erv[k], semE[k])

        def group_body(j, carry):
            cbase = NBUF * j
            for k in range(NBUF):
                pltpu.make_async_copy(
                    hx_hbm.at[srcall.at[cbase + k]], rows[k], semH[k]).wait()
                pltpu.make_async_copy(
                    er_hbm.at[dstall.at[cbase + k]], erv[k], semE[k]).wait()
                compute(rows[k], erv[k])
                pltpu.async_copy(rows[k], acc.at[dstall.at[cbase + k]],
                                 semS[k], add=True)
            for k in range(NBUF):
                pltpu.make_async_copy(
                    rows[k], acc.at[dstall.at[cbase + k]], semS[k]).wait()
                nxt = cbase + NBUF + k
                pltpu.async_copy(hx_hbm.at[srcall.at[nxt]], rows[k], semH[k])
                pltpu.async_copy(er_hbm.at[dstall.at[nxt]], erv[k], semE[k])
            return carry

        lax.fori_loop(0, CPW // NBUF, group_body, 0)
        for k in range(NBUF):
            pltpu.make_async_copy(
                hx_hbm.at[srcall.at[CPW + k]], rows[k], semH[k]).wait()
            pltpu.make_async_copy(
                er_hbm.at[dstall.at[CPW + k]], erv[k], semE[k]).wait()

        plsc.subcore_barrier()
        pltpu.sync_copy(acc.at[pl.ds(s * RPT, RPT)],
                        out_hbm.at[c, pl.ds(s * RPT, RPT)])

    return sc_edge


_sc_edge80 = _make_sc_edge(80, _edge_compute80)
_sc_edge16 = _make_sc_edge(16, _edge_compute16)


def kernel(x, edge_index, W0, al0, ar0, b0, W1, al1, ar1, b1,
           W2, al2, ar2, b2, Wres2, Wfc, bfc):
    E = edge_index.shape[1]
    pad_e = E_PAD - E
    pad_idx = (N + jnp.arange(pad_e, dtype=jnp.int32) % (NP_ - N)).astype(jnp.int32)
    src = jnp.concatenate([edge_index[0], pad_idx]).reshape(E_PAD // CHUNK, CHUNK)
    dst = jnp.concatenate([edge_index[1], pad_idx]).reshape(E_PAD // CHUNK, CHUNK)
    x_pad = jnp.pad(x, ((0, NP_ - N), (0, 0)))
    z80 = jnp.zeros((NP_, 80), _F32)
    z16 = jnp.zeros((NP_, 16), _F32)

    hx0, er0 = _tc_call(
        _tc0_body,
        (jax.ShapeDtypeStruct((NP_, 80), _F32), jax.ShapeDtypeStruct((NP_, 16), _F32)),
        x_pad, W0, al0, ar0)
    p0 = _sc_edge80(hx0, er0, src, dst, z80)

    hx1, er1, h0t = _tc_call(
        _tc1_body,
        (jax.ShapeDtypeStruct((NP_, 80), _F32), jax.ShapeDtypeStruct((NP_, 16), _F32),
         jax.ShapeDtypeStruct((NP_, 64), _F32)),
        p0, b0, W1, al1, ar1)
    p1 = _sc_edge80(hx1, er1, src, dst, z80)

    hx2, er2, res2 = _tc_call(
        _tc2_body,
        (jax.ShapeDtypeStruct((NP_, 16), _F32), jax.ShapeDtypeStruct((NP_, 16), _F32),
         jax.ShapeDtypeStruct((NP_, 8), _F32)),
        p1, h0t, b1, W2, Wres2, al2, ar2)
    p2 = _sc_edge16(hx2, er2, src, dst, z16)

    logits = _tc_call(
        _tc3_body,
        jax.ShapeDtypeStruct((NP_, NC), _F32),
        p2, res2, b2, Wfc, bfc)
    return logits[:N]
